# Initial kernel scaffold; baseline (speedup 1.0000x reference)
import jax
import jax.numpy as jnp
from jax.experimental import pallas as pl


def kernel(sequence, pos_num, adj_mask, adj_mat, token_table, pos_table, W_h, W_a, bias):
    raise NotImplementedError("write your pallas kernel here")


# speedup vs baseline: 2.6244x; 2.6244x over previous
import functools

import jax
import jax.numpy as jnp
from jax import lax
from jax.experimental import pallas as pl
from jax.experimental.pallas import tpu as pltpu
from jax.experimental.pallas import tpu_sc as plsc

B, L, V, E = 1024, 200, 1000, 128
NC, NS = 2, 16
NW = NC * NS
BPW = B // NW
LH = L // 2
BB = 8
EG = E // 16


def _adj_tc_kernel(adj_ref, mask_ref, wa_ref, wh_ref, bias_ref,
                   out_ref, mrep_ref):
    wa = wa_ref[...]
    rows = []
    for j in range(BB):
        rows.append(jnp.sum(adj_ref[j] * wa, axis=0, keepdims=True))
    v = jnp.concatenate(rows, axis=0)
    out_ref[...] = (
        jnp.dot(v, wh_ref[...], preferred_element_type=jnp.float32)
        + bias_ref[...]
    )
    mrep_ref[...] = jnp.broadcast_to(mask_ref[...][:, :, None], (BB, L, 16))


def _adj_out(adj_mat, adj_mask, W_a, W_h, bias):
    return pl.pallas_call(
        _adj_tc_kernel,
        grid=(B // BB,),
        in_specs=[
            pl.BlockSpec((BB, L, L), lambda i: (i, 0, 0)),
            pl.BlockSpec((BB, L), lambda i: (i, 0)),
            pl.BlockSpec((L, 1), lambda i: (0, 0)),
            pl.BlockSpec((L, E), lambda i: (0, 0)),
            pl.BlockSpec((1, E), lambda i: (0, 0)),
        ],
        out_specs=[
            pl.BlockSpec((BB, E), lambda i: (i, 0)),
            pl.BlockSpec((BB, L, 16), lambda i: (i, 0, 0)),
        ],
        out_shape=[
            jax.ShapeDtypeStruct((B, E), jnp.float32),
            jax.ShapeDtypeStruct((B, L, 16), jnp.float32),
        ],
    )(adj_mat, adj_mask, W_a.reshape(L, 1), W_h, bias.reshape(1, E))


def _sc_body(seq_hbm, pos_hbm, mrep_hbm, outv_hbm, tok_hbm, post_hbm, x_hbm,
             seq_v, pos_v, mrep_v, outv_v, tokbuf, posbuf, sem):
    wid = lax.axis_index("s") * NC + lax.axis_index("c")
    b0 = wid * BPW
    pltpu.sync_copy(seq_hbm.at[wid], seq_v)
    pltpu.sync_copy(pos_hbm.at[wid], pos_v)
    pltpu.sync_copy(outv_hbm.at[pl.ds(b0, BPW)], outv_v)

    def batch_body(i, carry):
        cps = [
            pltpu.async_copy(tok_hbm.at[seq_v.at[2 * i]],
                             tokbuf.at[pl.ds(0, LH)], sem),
            pltpu.async_copy(tok_hbm.at[seq_v.at[2 * i + 1]],
                             tokbuf.at[pl.ds(LH, LH)], sem),
            pltpu.async_copy(post_hbm.at[pos_v.at[2 * i]],
                             posbuf.at[pl.ds(0, LH)], sem),
            pltpu.async_copy(post_hbm.at[pos_v.at[2 * i + 1]],
                             posbuf.at[pl.ds(LH, LH)], sem),
            pltpu.async_copy(mrep_hbm.at[b0 + i], mrep_v, sem),
        ]
        for c in cps:
            c.wait()
        out_gs = [outv_v[i, pl.ds(g * 16, 16)] for g in range(EG)]

        def row(l, c2):
            ms = mrep_v[l, :]
            for g in range(EG):
                sl = pl.ds(g * 16, 16)
                tokbuf[l, sl] = tokbuf[l, sl] + posbuf[l, sl] + ms * out_gs[g]
            return c2

        lax.fori_loop(0, L, row, 0)
        pltpu.sync_copy(tokbuf, x_hbm.at[b0 + i])
        return carry

    lax.fori_loop(0, BPW, batch_body, 0)


_sc_gather = functools.partial(
    pl.kernel,
    out_type=jax.ShapeDtypeStruct((B, L, E), jnp.float32),
    mesh=plsc.VectorSubcoreMesh(core_axis_name="c", subcore_axis_name="s"),
    scratch_types=[
        pltpu.VMEM((2 * BPW, LH), jnp.int32),
        pltpu.VMEM((2 * BPW, LH), jnp.int32),
        pltpu.VMEM((L, 16), jnp.float32),
        pltpu.VMEM((BPW, E), jnp.float32),
        pltpu.VMEM((L, E), jnp.float32),
        pltpu.VMEM((L, E), jnp.float32),
        pltpu.SemaphoreType.DMA,
    ],
)(_sc_body)


def kernel(sequence, pos_num, adj_mask, adj_mat, token_table, pos_table,
           W_h, W_a, bias):
    outv, mrep = _adj_out(adj_mat, adj_mask, W_a, W_h, bias)
    seq_r = sequence.reshape(NW, 2 * BPW, LH)
    pos_r = pos_num.reshape(NW, 2 * BPW, LH)
    return _sc_gather(seq_r, pos_r, mrep, outv, token_table, pos_table)


# speedup vs baseline: 3.5521x; 1.3535x over previous
import functools

import jax
import jax.numpy as jnp
from jax import lax
from jax.experimental import pallas as pl
from jax.experimental.pallas import tpu as pltpu
from jax.experimental.pallas import tpu_sc as plsc

B, L, V, E = 1024, 200, 1000, 128
NC, NS = 2, 16
NW = NC * NS
BPW = B // NW
LH = L // 2
BB = 8
EG = E // 16


KB = 8
BC = 128


def _adjT_kernel(adjT_ref, wa_ref, vT_ref):
    blk = adjT_ref[...].reshape(L, KB * BC)
    res = jnp.dot(wa_ref[...], blk, preferred_element_type=jnp.float32)
    vT_ref[...] = res.reshape(KB, BC)


def _vT(adjT, W_a):
    return pl.pallas_call(
        _adjT_kernel,
        grid=(L // KB, B // BC),
        in_specs=[
            pl.BlockSpec((L, KB, BC), lambda ik, ib: (0, ik, ib)),
            pl.BlockSpec((1, L), lambda ik, ib: (0, 0)),
        ],
        out_specs=pl.BlockSpec((KB, BC), lambda ik, ib: (ik, ib)),
        out_shape=jax.ShapeDtypeStruct((L, B), jnp.float32),
    )(adjT, W_a.reshape(1, L))


def _outv_kernel(vT_ref, wh_ref, bias_ref, mask_ref, out_ref, mrep_ref):
    out_ref[...] = lax.dot_general(
        vT_ref[...], wh_ref[...], (((0,), (0,)), ((), ())),
        preferred_element_type=jnp.float32) + bias_ref[...]
    mrep_ref[...] = jnp.broadcast_to(mask_ref[...][:, :, None], (BC, L, 16))


def _adj_out(adj_mat, adj_mask, W_a, W_h, bias):
    adjT = jnp.transpose(adj_mat, (1, 2, 0))
    vT = _vT(adjT, W_a)
    return pl.pallas_call(
        _outv_kernel,
        grid=(B // BC,),
        in_specs=[
            pl.BlockSpec((L, BC), lambda ib: (0, ib)),
            pl.BlockSpec((L, E), lambda ib: (0, 0)),
            pl.BlockSpec((1, E), lambda ib: (0, 0)),
            pl.BlockSpec((BC, L), lambda ib: (ib, 0)),
        ],
        out_specs=[
            pl.BlockSpec((BC, E), lambda ib: (ib, 0)),
            pl.BlockSpec((BC, L, 16), lambda ib: (ib, 0, 0)),
        ],
        out_shape=[
            jax.ShapeDtypeStruct((B, E), jnp.float32),
            jax.ShapeDtypeStruct((B, L, 16), jnp.float32),
        ],
    )(vT, W_h, bias.reshape(1, E), adj_mask)


def _sc_body(seq_hbm, pos_hbm, mrep_hbm, outv_hbm, tok_hbm, post_hbm, x_hbm,
             seq_v, pos_v, mrep_v, outv_v, tokbuf, posbuf, sem):
    wid = lax.axis_index("s") * NC + lax.axis_index("c")
    b0 = wid * BPW
    pltpu.sync_copy(seq_hbm.at[wid], seq_v)
    pltpu.sync_copy(pos_hbm.at[wid], pos_v)
    pltpu.sync_copy(outv_hbm.at[pl.ds(b0, BPW)], outv_v)

    def batch_body(i, carry):
        cps = [
            pltpu.async_copy(tok_hbm.at[seq_v.at[2 * i]],
                             tokbuf.at[pl.ds(0, LH)], sem),
            pltpu.async_copy(tok_hbm.at[seq_v.at[2 * i + 1]],
                             tokbuf.at[pl.ds(LH, LH)], sem),
            pltpu.async_copy(post_hbm.at[pos_v.at[2 * i]],
                             posbuf.at[pl.ds(0, LH)], sem),
            pltpu.async_copy(post_hbm.at[pos_v.at[2 * i + 1]],
                             posbuf.at[pl.ds(LH, LH)], sem),
            pltpu.async_copy(mrep_hbm.at[b0 + i], mrep_v, sem),
        ]
        for c in cps:
            c.wait()
        out_gs = [outv_v[i, pl.ds(g * 16, 16)] for g in range(EG)]

        def row(l, c2):
            ms = mrep_v[l, :]
            for g in range(EG):
                sl = pl.ds(g * 16, 16)
                tokbuf[l, sl] = tokbuf[l, sl] + posbuf[l, sl] + ms * out_gs[g]
            return c2

        lax.fori_loop(0, L, row, 0)
        pltpu.sync_copy(tokbuf, x_hbm.at[b0 + i])
        return carry

    lax.fori_loop(0, BPW, batch_body, 0)


_sc_gather = functools.partial(
    pl.kernel,
    out_type=jax.ShapeDtypeStruct((B, L, E), jnp.float32),
    mesh=plsc.VectorSubcoreMesh(core_axis_name="c", subcore_axis_name="s"),
    scratch_types=[
        pltpu.VMEM((2 * BPW, LH), jnp.int32),
        pltpu.VMEM((2 * BPW, LH), jnp.int32),
        pltpu.VMEM((L, 16), jnp.float32),
        pltpu.VMEM((BPW, E), jnp.float32),
        pltpu.VMEM((L, E), jnp.float32),
        pltpu.VMEM((L, E), jnp.float32),
        pltpu.SemaphoreType.DMA,
    ],
)(_sc_body)


def kernel(sequence, pos_num, adj_mask, adj_mat, token_table, pos_table,
           W_h, W_a, bias):
    outv, mrep = _adj_out(adj_mat, adj_mask, W_a, W_h, bias)
    seq_r = sequence.reshape(NW, 2 * BPW, LH)
    pos_r = pos_num.reshape(NW, 2 * BPW, LH)
    return _sc_gather(seq_r, pos_r, mrep, outv, token_table, pos_table)
